# baseline (speedup 1.0000x reference)
import functools

import jax
import jax.numpy as jnp
from jax import lax
from jax.experimental import pallas as pl
from jax.experimental.pallas import tpu as pltpu
from jax.experimental.pallas import tpu_sc as plsc

_NC = 2
_NS = 16
_NW = _NC * _NS
_STAGERS = 10
_STAGE_ROUND = 10_000
_SPLIT = 15_360


def _sc_gather(idx_flat, values):
    total = idx_flat.shape[0]
    nvals = values.shape[0]
    assert total % (8 * _NW) == 0
    per_w = total // _NW
    stage_per = nvals // _STAGERS
    assert stage_per % _STAGE_ROUND == 0 and _STAGE_ROUND % 8 == 0
    nrounds = stage_per // _STAGE_ROUND
    h = _SPLIT
    assert h % 8 == 0 and 0 < h < per_w and _STAGE_ROUND <= h
    mesh = plsc.VectorSubcoreMesh(core_axis_name="c", subcore_axis_name="s")

    @functools.partial(
        pl.kernel,
        mesh=mesh,
        out_type=jax.ShapeDtypeStruct((total,), jnp.float32),
        scratch_types=[
            pltpu.VMEM_SHARED((nvals,), jnp.float32),
            pltpu.VMEM((per_w,), jnp.int32),
            pltpu.VMEM((per_w,), jnp.float32),
            pltpu.SemaphoreType.DMA,
            pltpu.SemaphoreType.DMA,
        ],
    )
    def k(idx_hbm, values_hbm, out_hbm, shared, idx_v, rows_v, hsem, ssem):
        c = lax.axis_index("c")
        s = lax.axis_index("s")
        wid = s * _NC + c
        base = wid * per_w
        pltpu.sync_copy(idx_hbm.at[pl.ds(base, per_w)], idx_v)
        hbm_cp = pltpu.async_copy(
            values_hbm.at[idx_v.at[pl.ds(h, per_w - h)]],
            rows_v.at[pl.ds(h, per_w - h)], hsem)

        @pl.when(s < _STAGERS)
        def _stage():
            bounce = rows_v.at[pl.ds(0, _STAGE_ROUND)]
            for j in range(nrounds):
                off = s * stage_per + j * _STAGE_ROUND
                pltpu.sync_copy(values_hbm.at[pl.ds(off, _STAGE_ROUND)],
                                bounce)
                pltpu.sync_copy(bounce, shared.at[pl.ds(off, _STAGE_ROUND)])

        plsc.subcore_barrier()
        pltpu.async_copy(shared.at[idx_v.at[pl.ds(0, h)]],
                         rows_v.at[pl.ds(0, h)], ssem).wait()
        hbm_cp.wait()
        pltpu.sync_copy(rows_v, out_hbm.at[pl.ds(base, per_w)])

    return k(idx_flat, values)


def kernel(indices, values):
    b, l = indices.shape
    idx_flat = indices.reshape(-1).astype(jnp.int32)
    out = _sc_gather(idx_flat, values)
    return out.reshape(b, l, 1)


# speedup vs baseline: 1.4136x; 1.4136x over previous
import functools

import jax
import jax.numpy as jnp
from jax import lax
from jax.experimental import pallas as pl
from jax.experimental.pallas import tpu as pltpu
from jax.experimental.pallas import tpu_sc as plsc

_NC = 2
_NS = 16
_NW = _NC * _NS
_STAGERS = 10
_STAGE_ROUND = 25_000


def _sc_gather(idx_flat, values):
    total = idx_flat.shape[0]
    nvals = values.shape[0]
    assert total % (8 * _NW) == 0
    per_w = total // _NW
    stage_per = nvals // _STAGERS
    assert stage_per % _STAGE_ROUND == 0 and _STAGE_ROUND % 8 == 0
    nrounds = stage_per // _STAGE_ROUND
    mesh = plsc.VectorSubcoreMesh(core_axis_name="c", subcore_axis_name="s")

    @functools.partial(
        pl.kernel,
        mesh=mesh,
        out_type=jax.ShapeDtypeStruct((total,), jnp.float32),
        scratch_types=[
            pltpu.VMEM_SHARED((nvals,), jnp.float32),
            pltpu.VMEM((per_w,), jnp.int32),
            pltpu.VMEM((per_w,), jnp.float32),
            pltpu.SemaphoreType.DMA,
            pltpu.SemaphoreType.DMA,
        ],
    )
    def k(idx_hbm, values_hbm, out_hbm, shared, idx_v, rows_v, sem, isem):
        c = lax.axis_index("c")
        s = lax.axis_index("s")
        wid = s * _NC + c
        base = wid * per_w
        idx_cp = pltpu.async_copy(idx_hbm.at[pl.ds(base, per_w)], idx_v, isem)

        @pl.when(s < _STAGERS)
        def _stage():
            bounce = rows_v.at[pl.ds(0, _STAGE_ROUND)]
            for j in range(nrounds):
                off = s * stage_per + j * _STAGE_ROUND
                pltpu.sync_copy(values_hbm.at[pl.ds(off, _STAGE_ROUND)],
                                bounce)
                pltpu.sync_copy(bounce, shared.at[pl.ds(off, _STAGE_ROUND)])

        plsc.subcore_barrier()
        idx_cp.wait()
        pltpu.async_copy(shared.at[idx_v], rows_v, sem).wait()
        pltpu.sync_copy(rows_v, out_hbm.at[pl.ds(base, per_w)])

    return k(idx_flat, values)


def kernel(indices, values):
    b, l = indices.shape
    idx_flat = indices.T.reshape(-1).astype(jnp.int32)
    out = _sc_gather(idx_flat, values)
    return out.reshape(l, b).T[:, :, None]


# speedup vs baseline: 1.6854x; 1.1922x over previous
import functools

import jax
import jax.numpy as jnp
from jax import lax
from jax.experimental import pallas as pl
from jax.experimental.pallas import tpu as pltpu
from jax.experimental.pallas import tpu_sc as plsc

_NC = 2
_NS = 16
_NW = _NC * _NS
_STAGERS = 10
_STAGE_ROUND = 25_000


def _sc_gather(idx_flat, values):
    total = idx_flat.shape[0]
    nvals = values.shape[0]
    assert total % (8 * _NW) == 0
    per_w = total // _NW
    stage_per = nvals // _STAGERS
    assert stage_per % _STAGE_ROUND == 0 and _STAGE_ROUND % 8 == 0
    nrounds = stage_per // _STAGE_ROUND
    mesh = plsc.VectorSubcoreMesh(core_axis_name="c", subcore_axis_name="s")

    @functools.partial(
        pl.kernel,
        mesh=mesh,
        out_type=jax.ShapeDtypeStruct((total,), jnp.float32),
        scratch_types=[
            pltpu.VMEM_SHARED((nvals,), jnp.float32),
            pltpu.VMEM((per_w,), jnp.int32),
            pltpu.VMEM((per_w,), jnp.float32),
            pltpu.SemaphoreType.DMA,
            pltpu.SemaphoreType.DMA,
        ],
    )
    def k(idx_hbm, values_hbm, out_hbm, shared, idx_v, rows_v, sem, isem):
        c = lax.axis_index("c")
        s = lax.axis_index("s")
        wid = s * _NC + c
        base = wid * per_w
        idx_cp = pltpu.async_copy(idx_hbm.at[pl.ds(base, per_w)], idx_v, isem)

        @pl.when(s < _STAGERS)
        def _stage():
            bounce = rows_v.at[pl.ds(0, _STAGE_ROUND)]
            for j in range(nrounds):
                off = s * stage_per + j * _STAGE_ROUND
                pltpu.sync_copy(values_hbm.at[pl.ds(off, _STAGE_ROUND)],
                                bounce)
                pltpu.sync_copy(bounce, shared.at[pl.ds(off, _STAGE_ROUND)])

        plsc.subcore_barrier()
        idx_cp.wait()
        pltpu.async_copy(shared.at[idx_v], rows_v, sem).wait()
        pltpu.sync_copy(rows_v, out_hbm.at[pl.ds(base, per_w)])

    return k(idx_flat, values)


def kernel(indices, values):
    b, l = indices.shape
    idx_flat = indices.T.reshape(-1).astype(jnp.int32)
    out = _sc_gather(idx_flat, values)
    return out.reshape(l, b, 1).transpose(1, 0, 2)
